# baseline (speedup 1.0000x reference)
import jax
import jax.numpy as jnp
from jax.experimental import pallas as pl
from jax.experimental.pallas import tpu as pltpu

_BM = 512
_NBUF = 4


def _gcn_body(adj_hbm, x_ref, w1_ref, b1_ref, w2_ref, b2_ref, o_ref,
              bufs, s_scr, x1_ref, sems):
    n_rel = w1_ref.shape[0]
    n = x_ref.shape[0]
    hid = w1_ref.shape[2]
    mi_per = n // _BM
    nsteps = n_rel * mi_per
    inv_a = 1.0 / n_rel

    half = _BM // 2

    def start_copy(step):
        r = jax.lax.rem(step, nsteps)
        slot = jax.lax.rem(step, _NBUF)
        pltpu.make_async_copy(
            adj_hbm.at[pl.ds(r * _BM, half), :],
            bufs.at[slot, pl.ds(0, half)],
            sems.at[slot, 0],
        ).start()
        pltpu.make_async_copy(
            adj_hbm.at[pl.ds(r * _BM + half, half), :],
            bufs.at[slot, pl.ds(half, half)],
            sems.at[slot, 1],
        ).start()

    for i in range(_NBUF):
        start_copy(i)

    xb = x_ref[...].astype(jnp.bfloat16)
    for a in range(n_rel):
        s_scr[0, a] = (
            jnp.dot(xb, w1_ref[a].astype(jnp.bfloat16),
                    preferred_element_type=jnp.float32)
            * inv_a
        ).astype(jnp.bfloat16)

    x1_ref[...] = jnp.zeros_like(x1_ref)
    o_ref[...] = jnp.zeros_like(o_ref)

    def step_fn(step, _):
        p = step // nsteps
        r = jax.lax.rem(step, nsteps)
        a = r // mi_per
        m_base = jax.lax.rem(r, mi_per) * _BM
        slot = jax.lax.rem(step, _NBUF)

        pltpu.make_async_copy(
            adj_hbm.at[pl.ds(r * _BM, half), :],
            bufs.at[slot, pl.ds(0, half)],
            sems.at[slot, 0],
        ).wait()
        s_blk = s_scr[p, a]
        c0 = jnp.dot(
            bufs[slot, pl.ds(0, half)].astype(jnp.bfloat16),
            s_blk,
            preferred_element_type=jnp.float32,
        )
        pltpu.make_async_copy(
            adj_hbm.at[pl.ds(r * _BM + half, half), :],
            bufs.at[slot, pl.ds(half, half)],
            sems.at[slot, 1],
        ).wait()
        c1 = jnp.dot(
            bufs[slot, pl.ds(half, half)].astype(jnp.bfloat16),
            s_blk,
            preferred_element_type=jnp.float32,
        )
        contrib = jnp.concatenate([c0, c1], axis=0)

        @pl.when(p == 0)
        def _acc1():
            x1_ref[pl.ds(m_base, _BM), :] += contrib

        @pl.when(p == 1)
        def _acc2():
            o_ref[pl.ds(m_base, _BM), :] += contrib

        @pl.when(jnp.logical_and(p == 0, a == n_rel - 1))
        def _mid():
            x1_blk = jnp.maximum(
                x1_ref[pl.ds(m_base, _BM), :] + b1_ref[...], 0.0
            )
            for a2 in range(n_rel):
                s_scr[1, a2, pl.ds(m_base, _BM), :] = (
                    jnp.dot(x1_blk, w2_ref[a2],
                            preferred_element_type=jnp.float32)
                    * inv_a
                ).astype(jnp.bfloat16)

        @pl.when(step + _NBUF < 2 * nsteps)
        def _next():
            start_copy(step + _NBUF)

        return ()

    jax.lax.fori_loop(0, 2 * nsteps, step_fn, (), unroll=2)
    o_ref[...] = jnp.maximum(o_ref[...] + b2_ref[...], 0.0)


@jax.jit
def kernel(x, adjs, W1, b1, W2, b2):
    n_rel, n, _ = adjs.shape
    hid = W1.shape[2]

    return pl.pallas_call(
        _gcn_body,
        in_specs=[
            pl.BlockSpec(memory_space=pltpu.MemorySpace.HBM),
            pl.BlockSpec(memory_space=pltpu.MemorySpace.VMEM),
            pl.BlockSpec(memory_space=pltpu.MemorySpace.VMEM),
            pl.BlockSpec(memory_space=pltpu.MemorySpace.VMEM),
            pl.BlockSpec(memory_space=pltpu.MemorySpace.VMEM),
            pl.BlockSpec(memory_space=pltpu.MemorySpace.VMEM),
        ],
        out_specs=pl.BlockSpec(memory_space=pltpu.MemorySpace.VMEM),
        out_shape=jax.ShapeDtypeStruct((n, hid), jnp.float32),
        scratch_shapes=[
            pltpu.VMEM((_NBUF, _BM, n), jnp.float32),
            pltpu.VMEM((2, n_rel, n, hid), jnp.bfloat16),
            pltpu.VMEM((n, hid), jnp.float32),
            pltpu.SemaphoreType.DMA((_NBUF, 2)),
        ],
        compiler_params=pltpu.CompilerParams(
            vmem_limit_bytes=63 * 1024 * 1024,
            disable_bounds_checks=True,
            skip_device_barrier=True,
        ),
    )(adjs.reshape(n_rel * n, n), x, W1, b1.reshape(1, hid), W2,
      b2.reshape(1, hid))


# speedup vs baseline: 1.0057x; 1.0057x over previous
import jax
import jax.numpy as jnp
from jax.experimental import pallas as pl
from jax.experimental.pallas import tpu as pltpu

_BM = 512
_NBUF = 4


def _gcn_body(adj_hbm, x_ref, w1_ref, b1_ref, w2_ref, b2_ref, o_ref,
              bufs, s_scr, x1_ref, sems):
    n_rel = w1_ref.shape[0]
    n = x_ref.shape[0]
    hid = w1_ref.shape[2]
    mi_per = n // _BM
    nsteps = n_rel * mi_per
    inv_a = 1.0 / n_rel

    half = _BM // 2

    def start_copy(step):
        r = jax.lax.rem(step, nsteps)
        slot = jax.lax.rem(step, _NBUF)
        pltpu.make_async_copy(
            adj_hbm.at[pl.ds(r * _BM, half), :],
            bufs.at[slot, pl.ds(0, half)],
            sems.at[slot, 0],
        ).start()
        pltpu.make_async_copy(
            adj_hbm.at[pl.ds(r * _BM + half, half), :],
            bufs.at[slot, pl.ds(half, half)],
            sems.at[slot, 1],
        ).start()

    for i in range(_NBUF):
        start_copy(i)

    xb = x_ref[...].astype(jnp.bfloat16)
    for a in range(n_rel):
        s_scr[0, a] = (
            jnp.dot(xb, w1_ref[a].astype(jnp.bfloat16),
                    preferred_element_type=jnp.float32)
            * inv_a
        ).astype(jnp.bfloat16)

    x1_ref[...] = jnp.zeros_like(x1_ref)
    o_ref[...] = jnp.zeros_like(o_ref)

    def step_fn(step, _):
        p = step // nsteps
        r = jax.lax.rem(step, nsteps)
        a = r // mi_per
        m_base = jax.lax.rem(r, mi_per) * _BM
        slot = jax.lax.rem(step, _NBUF)

        pltpu.make_async_copy(
            adj_hbm.at[pl.ds(r * _BM, half), :],
            bufs.at[slot, pl.ds(0, half)],
            sems.at[slot, 0],
        ).wait()
        pltpu.make_async_copy(
            adj_hbm.at[pl.ds(r * _BM + half, half), :],
            bufs.at[slot, pl.ds(half, half)],
            sems.at[slot, 1],
        ).wait()

        contrib = jnp.dot(
            bufs[slot].astype(jnp.bfloat16),
            s_scr[p, a],
            preferred_element_type=jnp.float32,
        )

        @pl.when(p == 0)
        def _acc1():
            x1_ref[pl.ds(m_base, _BM), :] += contrib

        @pl.when(p == 1)
        def _acc2():
            o_ref[pl.ds(m_base, _BM), :] += contrib

        @pl.when(jnp.logical_and(p == 0, a == n_rel - 1))
        def _mid():
            x1_blk = jnp.maximum(
                x1_ref[pl.ds(m_base, _BM), :] + b1_ref[...], 0.0
            )
            for a2 in range(n_rel):
                s_scr[1, a2, pl.ds(m_base, _BM), :] = (
                    jnp.dot(x1_blk, w2_ref[a2],
                            preferred_element_type=jnp.float32)
                    * inv_a
                ).astype(jnp.bfloat16)

        @pl.when(step + _NBUF < 2 * nsteps)
        def _next():
            start_copy(step + _NBUF)

        return ()

    jax.lax.fori_loop(0, 2 * nsteps, step_fn, (), unroll=2)
    o_ref[...] = jnp.maximum(o_ref[...] + b2_ref[...], 0.0)


@jax.jit
def kernel(x, adjs, W1, b1, W2, b2):
    n_rel, n, _ = adjs.shape
    hid = W1.shape[2]

    return pl.pallas_call(
        _gcn_body,
        in_specs=[
            pl.BlockSpec(memory_space=pltpu.MemorySpace.HBM),
            pl.BlockSpec(memory_space=pltpu.MemorySpace.VMEM),
            pl.BlockSpec(memory_space=pltpu.MemorySpace.VMEM),
            pl.BlockSpec(memory_space=pltpu.MemorySpace.VMEM),
            pl.BlockSpec(memory_space=pltpu.MemorySpace.VMEM),
            pl.BlockSpec(memory_space=pltpu.MemorySpace.VMEM),
        ],
        out_specs=pl.BlockSpec(memory_space=pltpu.MemorySpace.VMEM),
        out_shape=jax.ShapeDtypeStruct((n, hid), jnp.float32),
        scratch_shapes=[
            pltpu.VMEM((_NBUF, _BM, n), jnp.float32),
            pltpu.VMEM((2, n_rel, n, hid), jnp.bfloat16),
            pltpu.VMEM((n, hid), jnp.float32),
            pltpu.SemaphoreType.DMA((_NBUF, 2)),
        ],
        compiler_params=pltpu.CompilerParams(
            vmem_limit_bytes=63 * 1024 * 1024,
            disable_bounds_checks=True,
            skip_device_barrier=True,
        ),
    )(adjs.reshape(n_rel * n, n), x, W1, b1.reshape(1, hid), W2,
      b2.reshape(1, hid))
